# Initial kernel scaffold; baseline (speedup 1.0000x reference)
import jax
import jax.numpy as jnp
from jax.experimental import pallas as pl


def kernel(x, params):
    raise NotImplementedError("write your pallas kernel here")


# speedup vs baseline: 1.1200x; 1.1200x over previous
import functools

import jax
import jax.numpy as jnp
from jax.experimental import pallas as pl


def _first_argmin(d, cs):
    m = jnp.min(d, axis=1, keepdims=True)
    ii = jax.lax.broadcasted_iota(jnp.int32, d.shape, 1)
    idx = jnp.min(jnp.where(d == m, ii, cs), axis=1)
    onehot = (ii == idx[:, None]).astype(jnp.float32)
    return idx.astype(jnp.int32), onehot


def _vq_kernel(xp_ref, cinw_ref, cinb_ref, cbs_ref, cb2_ref, win_ref, bin_ref,
               wout_ref, bout_ref, q_ref, idx_ref, closs_ref,
               *, g, nq, cs, cd, dim, proj, tm, t, nct):
    c = pl.program_id(0)
    b = c // nct
    t0 = (c % nct) * tm

    @pl.when(c == 0)
    def _init():
        closs_ref[...] = jnp.zeros_like(closs_ref)

    xc = jnp.concatenate(
        [xp_ref[b, pl.ds(t0 + k, tm), :] for k in range(3)], axis=1)
    h = jnp.dot(xc, cinw_ref[...], preferred_element_type=jnp.float32)
    h = h + cinb_ref[0][None, :]

    dim_g = dim // g
    for gi in range(g):
        r = h[:, gi * dim_g:(gi + 1) * dim_g]
        qg = jnp.zeros((tm, dim_g), jnp.float32)
        for qi in range(nq):
            p = gi * nq + qi
            if proj:
                hh = jnp.dot(r, win_ref[p],
                             preferred_element_type=jnp.float32) + bin_ref[p][None, :]
            else:
                hh = r
            cbp = cbs_ref[p]
            cross = jax.lax.dot_general(
                hh, cbp, (((1,), (1,)), ((), ())),
                preferred_element_type=jnp.float32)
            d = (jnp.sum(hh * hh, axis=1, keepdims=True) - 2.0 * cross
                 + cb2_ref[p][None, :])
            idx, onehot = _first_argmin(d, cs)
            quant_cd = jax.lax.dot_general(
                onehot, cbp, (((1,), (0,)), ((), ())),
                preferred_element_type=jnp.float32,
                precision=jax.lax.Precision.HIGHEST)
            diff = quant_cd - hh
            ps = jnp.sum(diff * diff, axis=0)
            closs_ref[p, :] += jnp.sum(ps.reshape(cd // 128, 128), axis=0)
            quant_st = hh + diff
            if proj:
                quant = jnp.dot(quant_st, wout_ref[p],
                                preferred_element_type=jnp.float32) + bout_ref[p][None, :]
            else:
                quant = quant_st
            idx_ref[p, :] = idx
            r = r - quant
            qg = qg + quant
        q_ref[:, gi * dim_g:(gi + 1) * dim_g] = qg


def _conv_out_kernel(qp_ref, coutw_ref, coutb_ref, out_ref, *, t, in_dim):
    qc = jnp.concatenate(
        [qp_ref[0, pl.ds(k, t), :] for k in range(3)], axis=1)
    o = jnp.dot(qc, coutw_ref[...], preferred_element_type=jnp.float32)
    out_ref[0] = o + coutb_ref[0][None, :]


def _grvq_model(x, mp, *, tm=512):
    bsz, in_dim, t = x.shape
    groups = mp["groups"]
    g = len(groups)
    nq = len(groups[0])
    cs, cd = groups[0][0]["codebook"].shape
    dim = mp["conv_in_w"].shape[0]
    dim_g = dim // g
    proj = "w_in" in groups[0][0]
    p_total = g * nq
    nct = t // tm
    nchunks = bsz * nct
    bt = bsz * t

    xt = jnp.transpose(x, (0, 2, 1))
    xp = jnp.zeros((bsz, t + 8, in_dim), jnp.float32)
    xp = jax.lax.dynamic_update_slice(xp, xt, (0, 1, 0))
    cinw = jnp.concatenate(
        [mp["conv_in_w"][:, :, k].T for k in range(3)], axis=0)
    cinb = mp["conv_in_b"][None, :]
    cbs = jnp.stack([qp["codebook"] for ql in groups for qp in ql])
    cb2s = jnp.sum(cbs * cbs, axis=-1)
    if proj:
        win = jnp.stack([qp["w_in"] for ql in groups for qp in ql])
        binp = jnp.stack([qp["b_in"] for ql in groups for qp in ql])
        wout = jnp.stack([qp["w_out"] for ql in groups for qp in ql])
        bout = jnp.stack([qp["b_out"] for ql in groups for qp in ql])
    else:
        win = jnp.zeros((p_total, 8, 128), jnp.float32)
        binp = jnp.zeros((p_total, 128), jnp.float32)
        wout = jnp.zeros((p_total, 8, 128), jnp.float32)
        bout = jnp.zeros((p_total, 128), jnp.float32)

    full = lambda shape: pl.BlockSpec(shape, lambda c: (0,) * len(shape))
    q, idx, closs = pl.pallas_call(
        functools.partial(_vq_kernel, g=g, nq=nq, cs=cs, cd=cd, dim=dim,
                          proj=proj, tm=tm, t=t, nct=nct),
        grid=(nchunks,),
        in_specs=[
            full((bsz, t + 8, in_dim)),
            full((3 * in_dim, dim)),
            full((1, dim)),
            full((p_total, cs, cd)),
            full((p_total, cs)),
            full(win.shape),
            full(binp.shape),
            full(wout.shape),
            full(bout.shape),
        ],
        out_specs=[
            pl.BlockSpec((tm, dim), lambda c: (c, 0)),
            pl.BlockSpec((p_total, tm), lambda c: (0, c)),
            full((p_total, 128)),
        ],
        out_shape=[
            jax.ShapeDtypeStruct((bt, dim), jnp.float32),
            jax.ShapeDtypeStruct((p_total, bt), jnp.int32),
            jax.ShapeDtypeStruct((p_total, 128), jnp.float32),
        ],
    )(xp, cinw, cinb, cbs, cb2s, win, binp, wout, bout)

    qp_arr = jnp.zeros((bsz, t + 8, dim), jnp.float32)
    qp_arr = jax.lax.dynamic_update_slice(
        qp_arr, q.reshape(bsz, t, dim), (0, 1, 0))
    coutw = jnp.concatenate(
        [mp["conv_out_w"][:, :, k].T for k in range(3)], axis=0)
    coutb = mp["conv_out_b"][None, :]
    out_t = pl.pallas_call(
        functools.partial(_conv_out_kernel, t=t, in_dim=in_dim),
        grid=(bsz,),
        in_specs=[
            pl.BlockSpec((1, t + 8, dim), lambda b: (b, 0, 0)),
            pl.BlockSpec((3 * dim, in_dim), lambda b: (0, 0)),
            pl.BlockSpec((1, in_dim), lambda b: (0, 0)),
        ],
        out_specs=pl.BlockSpec((1, t, in_dim), lambda b: (b, 0, 0)),
        out_shape=jax.ShapeDtypeStruct((bsz, t, in_dim), jnp.float32),
    )(qp_arr, coutw, coutb)
    out = jnp.transpose(out_t, (0, 2, 1))

    indices = jnp.transpose(
        idx.reshape(g, nq, bsz, t), (0, 2, 3, 1))
    commit = (jnp.sum(closs, axis=1) / float(bt * cd)).reshape(g, nq)
    return out, indices, commit


def kernel(x, params):
    res = {}
    for name, mp in params.items():
        res[name] = _grvq_model(x, mp)
    return res


# speedup vs baseline: 1.1210x; 1.0009x over previous
import functools

import jax
import jax.numpy as jnp
from jax.experimental import pallas as pl


def _first_argmin(d, cs):
    m = jnp.min(d, axis=1, keepdims=True)
    ii = jax.lax.broadcasted_iota(jnp.int32, d.shape, 1)
    idx = jnp.min(jnp.where(d == m, ii, cs), axis=1)
    onehot = (ii == idx[:, None]).astype(jnp.float32)
    return idx.astype(jnp.int32), onehot


def _vq_kernel(xp_ref, cinw_ref, cinb_ref, cbs_ref, cb2_ref, win_ref, bin_ref,
               wout_ref, bout_ref, q_ref, idx_ref, closs_ref,
               *, g, nq, cs, cd, dim, proj, tm, t, nct):
    c = pl.program_id(0)
    b = c // nct
    t0 = (c % nct) * tm

    @pl.when(c == 0)
    def _init():
        closs_ref[...] = jnp.zeros_like(closs_ref)

    xc = jnp.concatenate(
        [xp_ref[b, pl.ds(t0 + k, tm), :] for k in range(3)], axis=1)
    h = jnp.dot(xc, cinw_ref[...], preferred_element_type=jnp.float32)
    h = h + cinb_ref[0][None, :]

    dim_g = dim // g
    for gi in range(g):
        r = h[:, gi * dim_g:(gi + 1) * dim_g]
        qg = jnp.zeros((tm, dim_g), jnp.float32)
        for qi in range(nq):
            p = gi * nq + qi
            if proj:
                hh = jnp.dot(r, win_ref[p],
                             preferred_element_type=jnp.float32) + bin_ref[p][None, :]
            else:
                hh = r
            cross = jax.lax.dot_general(
                hh, cbs_ref[p], (((1,), (1,)), ((), ())),
                preferred_element_type=jnp.float32)
            d = (jnp.sum(hh * hh, axis=1, keepdims=True) - 2.0 * cross
                 + cb2_ref[p][None, :])
            idx, onehot = _first_argmin(d, cs)
            quant_cd = jax.lax.dot_general(
                onehot, cbs_ref[p], (((1,), (0,)), ((), ())),
                preferred_element_type=jnp.float32,
                precision=jax.lax.Precision.HIGHEST)
            diff = quant_cd - hh
            ps = jnp.sum(diff * diff, axis=0)
            closs_ref[p, :] += jnp.sum(ps.reshape(cd // 128, 128), axis=0)
            quant_st = hh + diff
            if proj:
                quant = jnp.dot(quant_st, wout_ref[p],
                                preferred_element_type=jnp.float32) + bout_ref[p][None, :]
            else:
                quant = quant_st
            idx_ref[p, :] = idx
            r = r - quant
            qg = qg + quant
        q_ref[:, gi * dim_g:(gi + 1) * dim_g] = qg


def _conv_out_kernel(qp_ref, coutw_ref, coutb_ref, out_ref, *, t, in_dim):
    qc = jnp.concatenate(
        [qp_ref[0, pl.ds(k, t), :] for k in range(3)], axis=1)
    o = jnp.dot(qc, coutw_ref[...], preferred_element_type=jnp.float32)
    out_ref[0] = o + coutb_ref[0][None, :]


def _grvq_model(x, mp, *, tm=512):
    bsz, in_dim, t = x.shape
    groups = mp["groups"]
    g = len(groups)
    nq = len(groups[0])
    cs, cd = groups[0][0]["codebook"].shape
    dim = mp["conv_in_w"].shape[0]
    dim_g = dim // g
    proj = "w_in" in groups[0][0]
    p_total = g * nq
    nct = t // tm
    nchunks = bsz * nct
    bt = bsz * t

    xt = jnp.transpose(x, (0, 2, 1))
    xp = jnp.zeros((bsz, t + 8, in_dim), jnp.float32)
    xp = jax.lax.dynamic_update_slice(xp, xt, (0, 1, 0))
    cinw = jnp.concatenate(
        [mp["conv_in_w"][:, :, k].T for k in range(3)], axis=0)
    cinb = mp["conv_in_b"][None, :]
    cbs = jnp.stack([qp["codebook"] for ql in groups for qp in ql])
    cb2s = jnp.sum(cbs * cbs, axis=-1)
    if proj:
        win = jnp.stack([qp["w_in"] for ql in groups for qp in ql])
        binp = jnp.stack([qp["b_in"] for ql in groups for qp in ql])
        wout = jnp.stack([qp["w_out"] for ql in groups for qp in ql])
        bout = jnp.stack([qp["b_out"] for ql in groups for qp in ql])
    else:
        win = jnp.zeros((p_total, 8, 128), jnp.float32)
        binp = jnp.zeros((p_total, 128), jnp.float32)
        wout = jnp.zeros((p_total, 8, 128), jnp.float32)
        bout = jnp.zeros((p_total, 128), jnp.float32)

    full = lambda shape: pl.BlockSpec(shape, lambda c: (0,) * len(shape))
    q, idx, closs = pl.pallas_call(
        functools.partial(_vq_kernel, g=g, nq=nq, cs=cs, cd=cd, dim=dim,
                          proj=proj, tm=tm, t=t, nct=nct),
        grid=(nchunks,),
        in_specs=[
            full((bsz, t + 8, in_dim)),
            full((3 * in_dim, dim)),
            full((1, dim)),
            full((p_total, cs, cd)),
            full((p_total, cs)),
            full(win.shape),
            full(binp.shape),
            full(wout.shape),
            full(bout.shape),
        ],
        out_specs=[
            pl.BlockSpec((tm, dim), lambda c: (c, 0)),
            pl.BlockSpec((p_total, tm), lambda c: (0, c)),
            full((p_total, 128)),
        ],
        out_shape=[
            jax.ShapeDtypeStruct((bt, dim), jnp.float32),
            jax.ShapeDtypeStruct((p_total, bt), jnp.int32),
            jax.ShapeDtypeStruct((p_total, 128), jnp.float32),
        ],
    )(xp, cinw, cinb, cbs, cb2s, win, binp, wout, bout)

    qp_arr = jnp.zeros((bsz, t + 8, dim), jnp.float32)
    qp_arr = jax.lax.dynamic_update_slice(
        qp_arr, q.reshape(bsz, t, dim), (0, 1, 0))
    coutw = jnp.concatenate(
        [mp["conv_out_w"][:, :, k].T for k in range(3)], axis=0)
    coutb = mp["conv_out_b"][None, :]
    out_t = pl.pallas_call(
        functools.partial(_conv_out_kernel, t=t, in_dim=in_dim),
        grid=(bsz,),
        in_specs=[
            pl.BlockSpec((1, t + 8, dim), lambda b: (b, 0, 0)),
            pl.BlockSpec((3 * dim, in_dim), lambda b: (0, 0)),
            pl.BlockSpec((1, in_dim), lambda b: (0, 0)),
        ],
        out_specs=pl.BlockSpec((1, t, in_dim), lambda b: (b, 0, 0)),
        out_shape=jax.ShapeDtypeStruct((bsz, t, in_dim), jnp.float32),
    )(qp_arr, coutw, coutb)
    out = jnp.transpose(out_t, (0, 2, 1))

    indices = jnp.transpose(
        idx.reshape(g, nq, bsz, t), (0, 2, 3, 1))
    commit = (jnp.sum(closs, axis=1) / float(bt * cd)).reshape(g, nq)
    return out, indices, commit


def kernel(x, params):
    res = {}
    for name, mp in params.items():
        res[name] = _grvq_model(x, mp)
    return res
